# baseline (speedup 1.0000x reference)
import jax
import jax.numpy as jnp
from jax.experimental import pallas as pl
from jax.experimental.pallas import tpu as pltpu

N = 8192
H = 128
W = 128
TANFOVX = 0.5
TANFOVY = 0.5
SCALE_MOD = 1.0
FX = W / (2.0 * TANFOVX)
FY = H / (2.0 * TANFOVY)

NQ = 16
QH = H // NQ


def _cov3d(scales, rotations):
    q = rotations / jnp.linalg.norm(rotations, axis=1, keepdims=True)
    r, x, y, z = q[:, 0], q[:, 1], q[:, 2], q[:, 3]
    R = jnp.stack([1 - 2 * (y * y + z * z), 2 * (x * y - r * z), 2 * (x * z + r * y),
                   2 * (x * y + r * z), 1 - 2 * (x * x + z * z), 2 * (y * z - r * x),
                   2 * (x * z - r * y), 2 * (y * z + r * x), 1 - 2 * (x * x + y * y)],
                  axis=1).reshape(-1, 3, 3)
    M = R * (scales * SCALE_MOD)[:, None, :]
    return M @ jnp.swapaxes(M, 1, 2)


def _project(means3D, opacities, scales, rotations):
    t = means3D
    depth = t[:, 2]
    visible = depth > 0.2
    tz = jnp.where(visible, depth, 1.0)
    limx = 1.3 * TANFOVX
    limy = 1.3 * TANFOVY
    tx = jnp.clip(t[:, 0] / tz, -limx, limx) * tz
    ty = jnp.clip(t[:, 1] / tz, -limy, limy) * tz
    cov3d = _cov3d(scales, rotations)
    Nn = t.shape[0]
    J = jnp.zeros((Nn, 2, 3), dtype=jnp.float32)
    J = J.at[:, 0, 0].set(FX / tz).at[:, 0, 2].set(-FX * tx / (tz * tz))
    J = J.at[:, 1, 1].set(FY / tz).at[:, 1, 2].set(-FY * ty / (tz * tz))
    cov2d = jnp.einsum('nij,njk,nlk->nil', J, cov3d, J)
    a = cov2d[:, 0, 0] + 0.3
    c_ = cov2d[:, 1, 1] + 0.3
    b = cov2d[:, 0, 1]
    det = a * c_ - b * b
    det_ok = det > 0
    det_s = jnp.where(det_ok, det, 1.0)
    conic_a = c_ / det_s
    conic_b = -b / det_s
    conic_c = a / det_s
    px = (t[:, 0] / (tz * TANFOVX) + 1.0) * 0.5 * W - 0.5
    py = (t[:, 1] / (tz * TANFOVY) + 1.0) * 0.5 * H - 0.5
    mid = 0.5 * (a + c_)
    lam1 = mid + jnp.sqrt(jnp.maximum(mid * mid - det_s, 0.1))
    radii = jnp.where(visible & det_ok, jnp.ceil(3.0 * jnp.sqrt(lam1)), 0.0).astype(jnp.int32)
    valid = visible & det_ok & (radii > 0)
    op = jnp.where(valid, opacities[:, 0], 0.0)
    return px, py, conic_a, conic_b, conic_c, op, depth, lam1, radii, valid


def _render_body(par_ref, qb_ref, color_ref, T_ref, o0_ref, o1_ref, o2_ref):
    T_ref[...] = jnp.ones((H, W), jnp.float32)
    o0_ref[...] = jnp.zeros((H, W), jnp.float32)
    o1_ref[...] = jnp.zeros((H, W), jnp.float32)
    o2_ref[...] = jnp.zeros((H, W), jnp.float32)

    def body(g, _):
        px = par_ref[0, g]
        py = par_ref[1, g]
        A = par_ref[2, g]
        B = par_ref[3, g]
        C = par_ref[4, g]
        op = par_ref[5, g]
        d = par_ref[6, g]
        f2 = par_ref[7, g]
        qlo = qb_ref[0, g]
        qhi = qb_ref[1, g]

        def qstep(r, __):
            base = pl.multiple_of(r * QH, QH)
            ys = (jax.lax.broadcasted_iota(jnp.int32, (QH, W), 0) + base
                  ).astype(jnp.float32)
            xs = jax.lax.broadcasted_iota(jnp.int32, (QH, W), 1).astype(jnp.float32)
            dx = xs - px
            dy = ys - py
            power = dx * (A * dx + B * dy) + C * (dy * dy)
            alpha = jnp.minimum(0.99, op * jnp.exp(power))
            alpha = jnp.where((power <= 0.0) & (alpha >= 1.0 / 255.0), alpha, 0.0)
            T = T_ref[pl.ds(base, QH), :]
            w = T * alpha
            o0_ref[pl.ds(base, QH), :] += w * d
            o1_ref[pl.ds(base, QH), :] += w
            o2_ref[pl.ds(base, QH), :] += w * f2
            T_ref[pl.ds(base, QH), :] = T * (1.0 - alpha)
            return 0

        jax.lax.fori_loop(qlo, qhi + 1, qstep, 0)
        return 0

    jax.lax.fori_loop(0, N, body, 0)
    color_ref[0] = o0_ref[...]
    color_ref[1] = o1_ref[...]
    color_ref[2] = o2_ref[...]


def kernel(means3D, means2D, opacities, scales, rotations):
    px, py, ca, cb, cc, op, depth, lam1, radii, valid = _project(
        means3D, opacities, scales, rotations)
    sortkey = jnp.where(valid, depth, jnp.inf)
    order = jnp.argsort(sortkey)
    f2 = 1.0 / (1.0 + jnp.maximum(depth, 0.0))
    pars = jnp.stack([px[order], py[order], (-0.5 * ca)[order], (-cb)[order],
                      (-0.5 * cc)[order], op[order], depth[order],
                      f2[order]])

    op_s = pars[5]
    py_s = pars[1]
    lam1_s = lam1[order]
    r_cut = jnp.sqrt(jnp.maximum(2.0 * lam1_s * jnp.log(255.0 * op_s), 0.0)) + 1.0
    never = op_s * 255.0 <= 1.0
    qlo = jnp.clip(jnp.floor((py_s - r_cut) / QH), 0, NQ - 1).astype(jnp.int32)
    qhi = jnp.clip(jnp.floor((py_s + r_cut) / QH), 0, NQ - 1).astype(jnp.int32)
    offscreen = (py_s + r_cut < 0.0) | (py_s - r_cut > H - 1)
    skip = never | offscreen
    qlo = jnp.where(skip, 1, qlo)
    qhi = jnp.where(skip, 0, qhi)
    qb = jnp.stack([qlo, qhi])

    color = pl.pallas_call(
        _render_body,
        in_specs=[pl.BlockSpec(memory_space=pltpu.SMEM),
                  pl.BlockSpec(memory_space=pltpu.SMEM)],
        out_shape=jax.ShapeDtypeStruct((3, H, W), jnp.float32),
        scratch_shapes=[pltpu.VMEM((H, W), jnp.float32)] * 4,
    )(pars, qb)
    return color, radii


# speedup vs baseline: 1.0825x; 1.0825x over previous
import jax
import jax.numpy as jnp
from jax.experimental import pallas as pl
from jax.experimental.pallas import tpu as pltpu

N = 8192
H = 128
W = 128
TANFOVX = 0.5
TANFOVY = 0.5
SCALE_MOD = 1.0
FX = W / (2.0 * TANFOVX)
FY = H / (2.0 * TANFOVY)

NQ = 4
QH = H // NQ


def _cov3d(scales, rotations):
    q = rotations / jnp.linalg.norm(rotations, axis=1, keepdims=True)
    r, x, y, z = q[:, 0], q[:, 1], q[:, 2], q[:, 3]
    R = jnp.stack([1 - 2 * (y * y + z * z), 2 * (x * y - r * z), 2 * (x * z + r * y),
                   2 * (x * y + r * z), 1 - 2 * (x * x + z * z), 2 * (y * z - r * x),
                   2 * (x * z - r * y), 2 * (y * z + r * x), 1 - 2 * (x * x + y * y)],
                  axis=1).reshape(-1, 3, 3)
    M = R * (scales * SCALE_MOD)[:, None, :]
    return M @ jnp.swapaxes(M, 1, 2)


def _project(means3D, opacities, scales, rotations):
    t = means3D
    depth = t[:, 2]
    visible = depth > 0.2
    tz = jnp.where(visible, depth, 1.0)
    limx = 1.3 * TANFOVX
    limy = 1.3 * TANFOVY
    tx = jnp.clip(t[:, 0] / tz, -limx, limx) * tz
    ty = jnp.clip(t[:, 1] / tz, -limy, limy) * tz
    cov3d = _cov3d(scales, rotations)
    Nn = t.shape[0]
    J = jnp.zeros((Nn, 2, 3), dtype=jnp.float32)
    J = J.at[:, 0, 0].set(FX / tz).at[:, 0, 2].set(-FX * tx / (tz * tz))
    J = J.at[:, 1, 1].set(FY / tz).at[:, 1, 2].set(-FY * ty / (tz * tz))
    cov2d = jnp.einsum('nij,njk,nlk->nil', J, cov3d, J)
    a = cov2d[:, 0, 0] + 0.3
    c_ = cov2d[:, 1, 1] + 0.3
    b = cov2d[:, 0, 1]
    det = a * c_ - b * b
    det_ok = det > 0
    det_s = jnp.where(det_ok, det, 1.0)
    conic_a = c_ / det_s
    conic_b = -b / det_s
    conic_c = a / det_s
    px = (t[:, 0] / (tz * TANFOVX) + 1.0) * 0.5 * W - 0.5
    py = (t[:, 1] / (tz * TANFOVY) + 1.0) * 0.5 * H - 0.5
    mid = 0.5 * (a + c_)
    lam1 = mid + jnp.sqrt(jnp.maximum(mid * mid - det_s, 0.1))
    radii = jnp.where(visible & det_ok, jnp.ceil(3.0 * jnp.sqrt(lam1)), 0.0).astype(jnp.int32)
    valid = visible & det_ok & (radii > 0)
    op = jnp.where(valid, opacities[:, 0], 0.0)
    return px, py, conic_a, conic_b, conic_c, op, depth, lam1, radii, valid


def _render_body(par_ref, qb_ref, color_ref, T_ref, o0_ref, o1_ref, o2_ref):
    T_ref[...] = jnp.ones((H, W), jnp.float32)
    o0_ref[...] = jnp.zeros((H, W), jnp.float32)
    o1_ref[...] = jnp.zeros((H, W), jnp.float32)
    o2_ref[...] = jnp.zeros((H, W), jnp.float32)

    def body(g, _):
        px = par_ref[0, g]
        py = par_ref[1, g]
        A = par_ref[2, g]
        B = par_ref[3, g]
        C = par_ref[4, g]
        op = par_ref[5, g]
        d = par_ref[6, g]
        f2 = par_ref[7, g]
        qlo = qb_ref[0, g]
        qhi = qb_ref[1, g]

        def qstep(r, __):
            base = pl.multiple_of(r * QH, QH)
            ys = (jax.lax.broadcasted_iota(jnp.int32, (QH, W), 0) + base
                  ).astype(jnp.float32)
            xs = jax.lax.broadcasted_iota(jnp.int32, (QH, W), 1).astype(jnp.float32)
            dx = xs - px
            dy = ys - py
            power = dx * (A * dx + B * dy) + C * (dy * dy)
            alpha = jnp.minimum(0.99, op * jnp.exp(power))
            alpha = jnp.where((power <= 0.0) & (alpha >= 1.0 / 255.0), alpha, 0.0)
            T = T_ref[pl.ds(base, QH), :]
            w = T * alpha
            o0_ref[pl.ds(base, QH), :] += w * d
            o1_ref[pl.ds(base, QH), :] += w
            o2_ref[pl.ds(base, QH), :] += w * f2
            T_ref[pl.ds(base, QH), :] = T * (1.0 - alpha)
            return 0

        jax.lax.fori_loop(qlo, qhi + 1, qstep, 0)
        return 0

    jax.lax.fori_loop(0, N, body, 0)
    color_ref[0] = o0_ref[...]
    color_ref[1] = o1_ref[...]
    color_ref[2] = o2_ref[...]


def kernel(means3D, means2D, opacities, scales, rotations):
    px, py, ca, cb, cc, op, depth, lam1, radii, valid = _project(
        means3D, opacities, scales, rotations)
    sortkey = jnp.where(valid, depth, jnp.inf)
    order = jnp.argsort(sortkey)
    f2 = 1.0 / (1.0 + jnp.maximum(depth, 0.0))
    pars = jnp.stack([px[order], py[order], (-0.5 * ca)[order], (-cb)[order],
                      (-0.5 * cc)[order], op[order], depth[order],
                      f2[order]])

    op_s = pars[5]
    py_s = pars[1]
    lam1_s = lam1[order]
    r_cut = jnp.sqrt(jnp.maximum(2.0 * lam1_s * jnp.log(255.0 * op_s), 0.0)) + 1.0
    never = op_s * 255.0 <= 1.0
    qlo = jnp.clip(jnp.floor((py_s - r_cut) / QH), 0, NQ - 1).astype(jnp.int32)
    qhi = jnp.clip(jnp.floor((py_s + r_cut) / QH), 0, NQ - 1).astype(jnp.int32)
    offscreen = (py_s + r_cut < 0.0) | (py_s - r_cut > H - 1)
    skip = never | offscreen
    qlo = jnp.where(skip, 1, qlo)
    qhi = jnp.where(skip, 0, qhi)
    qb = jnp.stack([qlo, qhi])

    color = pl.pallas_call(
        _render_body,
        in_specs=[pl.BlockSpec(memory_space=pltpu.SMEM),
                  pl.BlockSpec(memory_space=pltpu.SMEM)],
        out_shape=jax.ShapeDtypeStruct((3, H, W), jnp.float32),
        scratch_shapes=[pltpu.VMEM((H, W), jnp.float32)] * 4,
    )(pars, qb)
    return color, radii
